# baseline (speedup 1.0000x reference)
import jax
import jax.numpy as jnp
from jax.experimental import pallas as pl
from jax.experimental.pallas import tpu as pltpu

_N_SIZE = 4
_CELL = 1.0


def _col(b_ref):
    return jnp.transpose(b_ref[:], (1, 0))


def _social_lstm_body(bounds_ref, x_ref, pm_ref, h0_ref, c0_ref,
                      win_ref, bin_ref, wsoc_ref, bsoc_ref, wih_ref, whh_ref,
                      bih_ref, bhh_ref, wout_ref, bout_ref,
                      out_ref, h_scr, c_scr, cm_scr, r_scr, wcat_scr,
                      wsoc_scr, xr_scr):
    T = out_ref.shape[0]
    N = out_ref.shape[1]
    med = win_ref.shape[0]
    soc = wsoc_ref.shape[0]
    hid = whh_ref.shape[1]
    ncell = _N_SIZE * _N_SIZE
    half = _N_SIZE / 2.0

    out_ref[:] = jnp.zeros(out_ref.shape, out_ref.dtype)
    h_scr[:] = jnp.transpose(h0_ref[:], (1, 0))
    c_scr[:] = jnp.transpose(c0_ref[:], (1, 0))

    wsoc = wsoc_ref[:]
    b_eff = _col(bsoc_ref) + jnp.sum(wsoc, axis=1, keepdims=True)
    b_lstm = _col(bih_ref) + _col(bhh_ref)
    bout_c = _col(bout_ref)
    wout = wout_ref[:].astype(jnp.bfloat16)
    win = win_ref[:]
    wsoc_scr[:] = wsoc.astype(jnp.bfloat16)
    wcat_scr[:, 0:med + soc] = wih_ref[:].astype(jnp.bfloat16)
    wcat_scr[:, med + soc:] = whh_ref[:].astype(jnp.bfloat16)

    for f in range(T):
        xr_scr[f, :, :] = jnp.transpose(x_ref[f, :, 2:4], (1, 0))
    cx_c = x_ref[:, :, 2:3]
    cy_c = x_ref[:, :, 3:4]
    cx_r = xr_scr[:, 0:1, :]
    cy_r = xr_scr[:, 1:2, :]

    gx = (cx_c - cx_r) / _CELL
    gy = (cy_c - cy_r) / _CELL
    inb = (jnp.abs(gx) <= half) & (jnp.abs(gy) <= half)
    iota_l = jax.lax.broadcasted_iota(jnp.int32, (T, N, N), 2)
    iota_s = jax.lax.broadcasted_iota(jnp.int32, (T, N, N), 1)
    mask = inb & (iota_l != iota_s)
    ix = jnp.clip(jnp.floor(gx + half).astype(jnp.int32), 0, _N_SIZE - 1)
    iy = jnp.clip(jnp.floor(gy + half).astype(jnp.int32), 0, _N_SIZE - 1)
    cm_scr[:] = jnp.where(mask, ix * _N_SIZE + iy, ncell)

    bin_b = jnp.transpose(bin_ref[:], (1, 0))[None]
    r_scr[:] = jax.nn.relu(win[None, :, 0:1] * cx_r
                           + win[None, :, 1:2] * cy_r
                           + bin_b).astype(jnp.bfloat16)

    def step(f, carry):
        pmf = pm_ref[pl.ds(f, 1)][0]
        hT = h_scr[:]
        cT = c_scr[:]
        cmf = cm_scr[pl.ds(f, 1)][0]
        rT = r_scr[pl.ds(f, 1)][0]

        hT_bf = hT.astype(jnp.bfloat16)
        bl = jnp.concatenate(
            [(cmf == c_idx).astype(jnp.bfloat16) for c_idx in range(ncell)],
            axis=1)
        s_all = jnp.dot(hT_bf, bl,
                        preferred_element_type=jnp.float32).astype(
                            jnp.bfloat16)
        sstack = jnp.concatenate(
            [s_all[:, c_idx * N:(c_idx + 1) * N] for c_idx in range(ncell)],
            axis=0)
        eT = jax.nn.relu(
            jnp.dot(wsoc_scr[:], sstack,
                    preferred_element_type=jnp.float32) + b_eff)

        vcat = jnp.concatenate([rT, eT.astype(jnp.bfloat16), hT_bf], axis=0)
        gates = jnp.dot(wcat_scr[:], vcat,
                        preferred_element_type=jnp.float32) + b_lstm
        i_g = jax.nn.sigmoid(gates[0:hid])
        f_g = jax.nn.sigmoid(gates[hid:2 * hid])
        g_g = jnp.tanh(gates[2 * hid:3 * hid])
        o_g = jax.nn.sigmoid(gates[3 * hid:4 * hid])
        c2 = f_g * cT + i_g * g_g
        h2 = o_g * jnp.tanh(c2)

        outT = (jnp.dot(wout, h2.astype(jnp.bfloat16),
                        preferred_element_type=jnp.float32)
                + bout_c) * pmf
        out_ref[pl.ds(f, 1), :, :] = jnp.transpose(outT, (1, 0))[None]
        h_scr[:] = h2
        c_scr[:] = c2
        return carry

    lo = jnp.maximum(bounds_ref[0] + 1, 0)
    hi = jnp.minimum(bounds_ref[1] + 1, T)
    jax.lax.fori_loop(lo, hi, step, 0)


def kernel(X, part_masks, all_h_t, all_c_t, W_in, b_in, W_soc, b_soc,
           W_ih, W_hh, b_ih, b_hh, W_out, b_out, T_obs, T_pred):
    T, N = X.shape[0], X.shape[1]
    med = W_in.shape[0]
    soc = W_soc.shape[0]
    hid = W_hh.shape[1]
    out_dim = W_out.shape[0]

    bounds = jnp.stack([jnp.asarray(T_obs, jnp.int32),
                        jnp.asarray(T_pred, jnp.int32)])

    smem = pl.BlockSpec(memory_space=pltpu.SMEM)

    return pl.pallas_call(
        _social_lstm_body,
        out_shape=jax.ShapeDtypeStruct((T, N, out_dim), X.dtype),
        in_specs=[smem] + [pl.BlockSpec()] * 14,
        out_specs=pl.BlockSpec(),
        scratch_shapes=[pltpu.VMEM((hid, N), jnp.float32),
                        pltpu.VMEM((hid, N), jnp.float32),
                        pltpu.VMEM((T, N, N), jnp.int32),
                        pltpu.VMEM((T, med, N), jnp.bfloat16),
                        pltpu.VMEM((4 * hid, med + soc + hid), jnp.bfloat16),
                        pltpu.VMEM((soc, 16 * hid), jnp.bfloat16),
                        pltpu.VMEM((T, 2, N), jnp.float32)],
    )(bounds, X, part_masks, all_h_t, all_c_t,
      W_in, b_in.reshape(1, -1), W_soc, b_soc.reshape(1, -1),
      W_ih, W_hh, b_ih.reshape(1, -1), b_hh.reshape(1, -1),
      W_out, b_out.reshape(1, -1))


# speedup vs baseline: 1.0380x; 1.0380x over previous
import jax
import jax.numpy as jnp
from jax.experimental import pallas as pl
from jax.experimental.pallas import tpu as pltpu

_N_SIZE = 4
_CELL = 1.0


def _col(b_ref):
    return jnp.transpose(b_ref[:], (1, 0))


def _social_lstm_body(tob_ref, tpr_ref, x_ref, xt_ref, pm_ref, h0_ref, c0_ref,
                      win_ref, bin_ref, wsoc_ref, bsoc_ref, wih_ref, whh_ref,
                      bih_ref, bhh_ref, wout_ref, bout_ref,
                      out_ref, h_scr, c_scr, cm_scr, r_scr, wcat_scr,
                      wsoc_scr):
    T = out_ref.shape[0]
    N = out_ref.shape[1]
    med = win_ref.shape[0]
    soc = wsoc_ref.shape[0]
    hid = whh_ref.shape[1]
    ncell = _N_SIZE * _N_SIZE
    half = _N_SIZE / 2.0

    out_ref[:] = jnp.zeros(out_ref.shape, out_ref.dtype)
    h_scr[:] = jnp.transpose(h0_ref[:], (1, 0))
    c_scr[:] = jnp.transpose(c0_ref[:], (1, 0))

    wsoc = wsoc_ref[:]
    b_eff = _col(bsoc_ref) + jnp.sum(wsoc, axis=1, keepdims=True)
    b_lstm = _col(bih_ref) + _col(bhh_ref)
    bout_c = _col(bout_ref)
    wout = wout_ref[:].astype(jnp.bfloat16)
    win = win_ref[:]
    wsoc_scr[:] = wsoc.astype(jnp.bfloat16)
    wcat_scr[:, 0:med + soc] = wih_ref[:].astype(jnp.bfloat16)
    wcat_scr[:, med + soc:] = whh_ref[:].astype(jnp.bfloat16)

    cx_c = x_ref[:, :, 2:3]
    cy_c = x_ref[:, :, 3:4]
    cx_r = xt_ref[:, 2:3, :]
    cy_r = xt_ref[:, 3:4, :]

    gx = (cx_c - cx_r) / _CELL
    gy = (cy_c - cy_r) / _CELL
    inb = (jnp.abs(gx) <= half) & (jnp.abs(gy) <= half)
    iota_l = jax.lax.broadcasted_iota(jnp.int32, (T, N, N), 2)
    iota_s = jax.lax.broadcasted_iota(jnp.int32, (T, N, N), 1)
    mask = inb & (iota_l != iota_s)
    ix = jnp.clip(jnp.floor(gx + half).astype(jnp.int32), 0, _N_SIZE - 1)
    iy = jnp.clip(jnp.floor(gy + half).astype(jnp.int32), 0, _N_SIZE - 1)
    cm_scr[:] = jnp.where(mask, ix * _N_SIZE + iy, ncell)

    bin_b = jnp.transpose(bin_ref[:], (1, 0))[None]
    r_scr[:] = jax.nn.relu(win[None, :, 0:1] * cx_r
                           + win[None, :, 1:2] * cy_r
                           + bin_b).astype(jnp.bfloat16)

    def step(f, carry):
        pmf = pm_ref[pl.ds(f, 1)][0]
        hT = h_scr[:]
        cT = c_scr[:]
        cmf = cm_scr[pl.ds(f, 1)][0]
        rT = r_scr[pl.ds(f, 1)][0]

        hT_bf = hT.astype(jnp.bfloat16)
        bl = jnp.concatenate(
            [(cmf == c_idx).astype(jnp.bfloat16) for c_idx in range(ncell)],
            axis=1)
        s_all = jnp.dot(hT_bf, bl,
                        preferred_element_type=jnp.float32).astype(
                            jnp.bfloat16)
        sstack = jnp.concatenate(
            [s_all[:, c_idx * N:(c_idx + 1) * N] for c_idx in range(ncell)],
            axis=0)
        eT = jax.nn.relu(
            jnp.dot(wsoc_scr[:], sstack,
                    preferred_element_type=jnp.float32) + b_eff)

        vcat = jnp.concatenate([rT, eT.astype(jnp.bfloat16), hT_bf], axis=0)
        gates = jnp.dot(wcat_scr[:], vcat,
                        preferred_element_type=jnp.float32) + b_lstm
        i_g = jax.nn.sigmoid(gates[0:hid])
        f_g = jax.nn.sigmoid(gates[hid:2 * hid])
        g_g = jnp.tanh(gates[2 * hid:3 * hid])
        o_g = jax.nn.sigmoid(gates[3 * hid:4 * hid])
        c2 = f_g * cT + i_g * g_g
        h2 = o_g * jnp.tanh(c2)

        outT = (jnp.dot(wout, h2.astype(jnp.bfloat16),
                        preferred_element_type=jnp.float32)
                + bout_c) * pmf
        out_ref[pl.ds(f, 1), :, :] = jnp.transpose(outT, (1, 0))[None]
        h_scr[:] = h2
        c_scr[:] = c2
        return carry

    lo = jnp.maximum(tob_ref[0] + 1, 0)
    hi = jnp.minimum(tpr_ref[0] + 1, T)
    jax.lax.fori_loop(lo, hi, step, 0)


def kernel(X, part_masks, all_h_t, all_c_t, W_in, b_in, W_soc, b_soc,
           W_ih, W_hh, b_ih, b_hh, W_out, b_out, T_obs, T_pred):
    T, N = X.shape[0], X.shape[1]
    med = W_in.shape[0]
    soc = W_soc.shape[0]
    hid = W_hh.shape[1]
    out_dim = W_out.shape[0]

    xt = jnp.transpose(X, (0, 2, 1))
    tob = jnp.asarray(T_obs, jnp.int32).reshape(1)
    tpr = jnp.asarray(T_pred, jnp.int32).reshape(1)

    smem = pl.BlockSpec(memory_space=pltpu.SMEM)

    return pl.pallas_call(
        _social_lstm_body,
        out_shape=jax.ShapeDtypeStruct((T, N, out_dim), X.dtype),
        in_specs=[smem, smem] + [pl.BlockSpec()] * 15,
        out_specs=pl.BlockSpec(),
        scratch_shapes=[pltpu.VMEM((hid, N), jnp.float32),
                        pltpu.VMEM((hid, N), jnp.float32),
                        pltpu.VMEM((T, N, N), jnp.int32),
                        pltpu.VMEM((T, med, N), jnp.bfloat16),
                        pltpu.VMEM((4 * hid, med + soc + hid), jnp.bfloat16),
                        pltpu.VMEM((soc, 16 * hid), jnp.bfloat16)],
    )(tob, tpr, X, xt, part_masks, all_h_t, all_c_t,
      W_in, b_in.reshape(1, -1), W_soc, b_soc.reshape(1, -1),
      W_ih, W_hh, b_ih.reshape(1, -1), b_hh.reshape(1, -1),
      W_out, b_out.reshape(1, -1))
